# Initial kernel scaffold; baseline (speedup 1.0000x reference)
import jax
import jax.numpy as jnp
from jax.experimental import pallas as pl


def kernel(X, adj_mat, W0, W1, W2, b0, b1, b2, g0, g1, g2, beta0, beta1, beta2):
    raise NotImplementedError("write your pallas kernel here")


# speedup vs baseline: 1682.8720x; 1682.8720x over previous
import jax
import jax.numpy as jnp
from jax.experimental import pallas as pl

_L = 3
_EPS = 1e-5


def _gnn_body(x_ref, a_ref,
              w0_ref, w1_ref, w2_ref,
              b0_ref, b1_ref, b2_ref,
              g0_ref, g1_ref, g2_ref,
              t0_ref, t1_ref, t2_ref,
              o_ref):
    x = x_ref[0]
    a = a_ref[0]
    ws = (w0_ref, w1_ref, w2_ref)
    bs = (b0_ref, b1_ref, b2_ref)
    gs = (g0_ref, g1_ref, g2_ref)
    ts = (t0_ref, t1_ref, t2_ref)
    for li in range(_L):
        xw = jnp.dot(x, ws[li][...], preferred_element_type=jnp.float32)
        agg = jax.lax.dot_general(
            a, xw, (((0,), (0,)), ((), ())),
            preferred_element_type=jnp.float32)
        z = agg + bs[li][...] + x
        mu = jnp.mean(z, axis=-1, keepdims=True)
        zc = z - mu
        var = jnp.mean(zc * zc, axis=-1, keepdims=True)
        y = zc * jax.lax.rsqrt(var + _EPS) * gs[li][...] + ts[li][...]
        x = jnp.maximum(y, 0.0)
    o_ref[0] = x


def kernel(X, adj_mat, W0, W1, W2, b0, b1, b2, g0, g1, g2, beta0, beta1, beta2):
    B, N, D = X.shape
    vecs = [v.reshape(1, D) for v in (b0, b1, b2, g0, g1, g2, beta0, beta1, beta2)]
    full2d = pl.BlockSpec((D, D), lambda i: (0, 0))
    vec2d = pl.BlockSpec((1, D), lambda i: (0, 0))
    out = pl.pallas_call(
        _gnn_body,
        grid=(B,),
        in_specs=[
            pl.BlockSpec((1, N, D), lambda i: (i, 0, 0)),
            pl.BlockSpec((1, N, N), lambda i: (i, 0, 0)),
            full2d, full2d, full2d,
            vec2d, vec2d, vec2d,
            vec2d, vec2d, vec2d,
            vec2d, vec2d, vec2d,
        ],
        out_specs=pl.BlockSpec((1, N, D), lambda i: (i, 0, 0)),
        out_shape=jax.ShapeDtypeStruct((B, N, D), jnp.float32),
    )(X, adj_mat, W0, W1, W2, *vecs)
    return out
